# baseline (speedup 1.0000x reference)
import functools

import jax
import jax.numpy as jnp
from jax import lax
from jax.experimental import pallas as pl
from jax.experimental.pallas import tpu as pltpu
from jax.experimental.pallas import tpu_sc as plsc

_ALPHA = 1000
_SEQ = 16384
_NC, _NS = 2, 16
_NW = _NC * _NS
_COLS = _SEQ // _NW
_R = 200
_NCH = _ALPHA // _R
_NBUF = 1
_LANE = 16
_NV = _COLS // _LANE


def _body(seq_hbm, out_hbm, seqv, bufs, sem):
    cid = lax.axis_index("c")
    sid = lax.axis_index("s")
    wid = sid * _NC + cid
    cbase = pl.multiple_of(wid * _COLS, _COLS)
    iota = lax.iota(jnp.int32, _LANE)

    zero16 = jnp.zeros((_LANE,), jnp.float32)

    def zb(i, carry):
        row = i // _NV
        col = pl.multiple_of((i % _NV) * _LANE, _LANE)
        for b in range(_NBUF):
            bufs[b, row, pl.ds(col, _LANE)] = zero16
        return carry

    lax.fori_loop(0, _R * _NV, zb, 0)

    pltpu.sync_copy(seq_hbm.at[pl.ds(cbase, _COLS)], seqv)

    ones16 = jnp.ones((_LANE,), jnp.float32)

    def scan_chunk(r0, value):
        def sj(j, carry):
            c_local = j * _LANE + iota
            s = seqv[pl.ds(pl.multiple_of(j * _LANE, _LANE), _LANE)]
            mask = (s >= r0) & (s < r0 + _R)
            plsc.store_scatter(
                bufs.at[value[0]], [s - r0, c_local],
                jnp.full((_LANE,), value[1], jnp.float32), mask=mask
            )
            return carry
        lax.fori_loop(0, _NV, sj, 0)

    descs = [None] * _NCH
    for t in range(_NCH):
        b = t % _NBUF
        if t >= _NBUF:
            descs[t - _NBUF].wait()
            scan_chunk((t - _NBUF) * _R, (b, 0.0))
        scan_chunk(t * _R, (b, 1.0))
        descs[t] = pltpu.async_copy(
            bufs.at[b],
            out_hbm.at[pl.ds(t * _R, _R), pl.ds(cbase, _COLS)],
            sem,
        )
    for t in range(_NCH - _NBUF, _NCH):
        descs[t].wait()


@functools.partial(jax.jit, static_argnums=())
def _one_hot_sc(sequence):
    mesh = plsc.VectorSubcoreMesh(
        core_axis_name="c", subcore_axis_name="s", num_cores=_NC,
        num_subcores=_NS,
    )
    fn = pl.kernel(
        _body,
        out_type=jax.ShapeDtypeStruct((_ALPHA, _SEQ), jnp.float32),
        mesh=mesh,
        scratch_types=[
            pltpu.VMEM((_COLS,), jnp.int32),
            pltpu.VMEM((_NBUF, _R, _COLS), jnp.float32),
            pltpu.SemaphoreType.DMA,
        ],
        compiler_params=pltpu.CompilerParams(
            use_tc_tiling_on_sc=False, needs_layout_passes=False
        ),
    )
    return fn(sequence)


def kernel(sequence):
    return _one_hot_sc(sequence.astype(jnp.int32))


# speedup vs baseline: 1.1676x; 1.1676x over previous
import functools

import jax
import jax.numpy as jnp
from jax import lax
from jax.experimental import pallas as pl
from jax.experimental.pallas import tpu as pltpu
from jax.experimental.pallas import tpu_sc as plsc

_ALPHA = 1000
_SEQ = 16384
_NC, _NS = 2, 16
_NW = _NC * _NS
_COLS = _SEQ // _NW
_R = 40
_NCH = _ALPHA // _R
_NBUF = 4
_LANE = 16
_NV = _COLS // _LANE


def _body(seq_hbm, out_hbm, seqv, bufs, sem):
    cid = lax.axis_index("c")
    sid = lax.axis_index("s")
    wid = sid * _NC + cid
    cbase = pl.multiple_of(wid * _COLS, _COLS)
    iota = lax.iota(jnp.int32, _LANE)

    zero16 = jnp.zeros((_LANE,), jnp.float32)

    def zb(i, carry):
        row = i // _NV
        col = pl.multiple_of((i % _NV) * _LANE, _LANE)
        for b in range(_NBUF):
            bufs[b, row, pl.ds(col, _LANE)] = zero16
        return carry

    lax.fori_loop(0, _R * _NV, zb, 0)

    pltpu.sync_copy(seq_hbm.at[pl.ds(cbase, _COLS)], seqv)

    def scan_chunk(r0, value):
        def sj(j, carry):
            c_local = j * _LANE + iota
            s = seqv[pl.ds(pl.multiple_of(j * _LANE, _LANE), _LANE)]
            mask = (s >= r0) & (s < r0 + _R)
            plsc.store_scatter(
                bufs.at[value[0]], [s - r0, c_local],
                jnp.full((_LANE,), value[1], jnp.float32), mask=mask
            )
            return carry
        lax.fori_loop(0, _NV, sj, 0)

    descs = [None] * _NCH
    for t in range(_NCH):
        b = t % _NBUF
        if t >= _NBUF:
            descs[t - _NBUF].wait()
            scan_chunk((t - _NBUF) * _R, (b, 0.0))
        scan_chunk(t * _R, (b, 1.0))
        descs[t] = pltpu.async_copy(
            bufs.at[b],
            out_hbm.at[pl.ds(t * _R, _R), pl.ds(cbase, _COLS)],
            sem,
        )
    for t in range(_NCH - _NBUF, _NCH):
        descs[t].wait()


@functools.partial(jax.jit, static_argnums=())
def _one_hot_sc(sequence):
    mesh = plsc.VectorSubcoreMesh(
        core_axis_name="c", subcore_axis_name="s", num_cores=_NC,
        num_subcores=_NS,
    )
    fn = pl.kernel(
        _body,
        out_type=jax.ShapeDtypeStruct((_ALPHA, _SEQ), jnp.float32),
        mesh=mesh,
        scratch_types=[
            pltpu.VMEM((_COLS,), jnp.int32),
            pltpu.VMEM((_NBUF, _R, _COLS), jnp.float32),
            pltpu.SemaphoreType.DMA,
        ],
        compiler_params=pltpu.CompilerParams(
            use_tc_tiling_on_sc=False, needs_layout_passes=False
        ),
    )
    return fn(sequence)


def kernel(sequence):
    return _one_hot_sc(sequence.astype(jnp.int32))
